# baseline (speedup 1.0000x reference)
import jax
import jax.numpy as jnp
from jax.experimental import pallas as pl
from jax.experimental.pallas import tpu as pltpu

_NQ = 6
_NB = 1024
_D = 512
_TT = 256
_NTOK = 8192
_NT = _NTOK // _TT


def _rvq_body(x_ref, cb_ref, qout_ref, idx_ref, loss_ref, perp_ref,
              cnorm_ref, counts_ref, losssum_ref,
              chi_ref, cmid_ref, clo_ref, cm2_ref):
    i = pl.program_id(0)

    @pl.when(i == 0)
    def _init():
        for q in range(_NQ):
            cq = cb_ref[q]
            cnorm_ref[q, :] = jnp.sum(cq * cq, axis=1)
            hi = cq.astype(jnp.bfloat16)
            rem = cq - hi.astype(jnp.float32)
            mid = rem.astype(jnp.bfloat16)
            lo = rem - mid.astype(jnp.float32)
            chi_ref[q] = hi
            cmid_ref[q] = mid
            clo_ref[q] = lo.astype(jnp.bfloat16)
            cm2_ref[q] = -2.0 * cq
        counts_ref[...] = jnp.zeros_like(counts_ref)
        losssum_ref[...] = jnp.zeros_like(losssum_ref)

    x = x_ref[...]
    res = x
    lanes = jax.lax.broadcasted_iota(jnp.int32, (_TT, _NB), 1)
    ones8 = jnp.ones((8, _TT), jnp.bfloat16)
    for q in range(_NQ):
        cq = cb_ref[q]
        xnorm = jnp.sum(res * res, axis=1, keepdims=True)
        if q > 0:
            losssum_ref[q - 1, :] += jnp.sum(xnorm)
        prodm2 = jax.lax.dot_general(
            res, cm2_ref[q], (((1,), (1,)), ((), ())),
            preferred_element_type=jnp.float32)
        scores = (xnorm + prodm2) + cnorm_ref[q, :][None, :]
        m = jnp.min(scores, axis=1, keepdims=True)
        idx = jnp.min(jnp.where(scores == m, lanes, _NB), axis=1)
        oh = (lanes == idx[:, None]).astype(jnp.bfloat16)
        dn = (((1,), (0,)), ((), ()))
        quant = (jax.lax.dot_general(oh, chi_ref[q], dn,
                                     preferred_element_type=jnp.float32)
                 + jax.lax.dot_general(oh, cmid_ref[q], dn,
                                       preferred_element_type=jnp.float32)
                 + jax.lax.dot_general(oh, clo_ref[q], dn,
                                       preferred_element_type=jnp.float32))
        res = res - quant
        idx_ref[q, :] = idx
        cnt = jax.lax.dot_general(ones8, oh, (((1,), (0,)), ((), ())),
                                  preferred_element_type=jnp.float32)
        counts_ref[q, :] += cnt[0, :]

    losssum_ref[_NQ - 1, :] += jnp.sum(res * res)
    qout_ref[...] = x - res

    @pl.when(i == _NT - 1)
    def _fin():
        rowmask = (jax.lax.broadcasted_iota(jnp.int32, (8, 1), 0) < _NQ)
        rowmask = rowmask.astype(jnp.float32)
        probs = counts_ref[...] * (1.0 / _NTOK)
        ent = -jnp.sum(probs * jnp.log(probs + 1e-10), axis=1, keepdims=True)
        perp = jnp.exp(ent) * rowmask
        perp_ref[...] = jnp.full((1, 1), jnp.sum(perp) / _NQ, jnp.float32)
        lsum = losssum_ref[...][:, 0:1] * rowmask
        mean_loss = jnp.sum(lsum) * (1.0 / (_NTOK * _D)) / _NQ
        loss_ref[...] = jnp.full((1, 1), mean_loss, jnp.float32)


def kernel(x, codebooks):
    B, D, T = x.shape
    x_tok = x.transpose(0, 2, 1).reshape(B * T, D)
    qout, idx6, loss, perp = pl.pallas_call(
        _rvq_body,
        grid=(_NT,),
        in_specs=[
            pl.BlockSpec((_TT, _D), lambda i: (i, 0)),
            pl.BlockSpec((_NQ, _NB, _D), lambda i: (0, 0, 0)),
        ],
        out_specs=[
            pl.BlockSpec((_TT, _D), lambda i: (i, 0)),
            pl.BlockSpec((_NQ, _TT), lambda i: (0, i)),
            pl.BlockSpec((1, 1), lambda i: (0, 0)),
            pl.BlockSpec((1, 1), lambda i: (0, 0)),
        ],
        out_shape=[
            jax.ShapeDtypeStruct((_NTOK, _D), jnp.float32),
            jax.ShapeDtypeStruct((_NQ, _NTOK), jnp.int32),
            jax.ShapeDtypeStruct((1, 1), jnp.float32),
            jax.ShapeDtypeStruct((1, 1), jnp.float32),
        ],
        scratch_shapes=[
            pltpu.VMEM((8, _NB), jnp.float32),
            pltpu.VMEM((8, _NB), jnp.float32),
            pltpu.VMEM((8, 128), jnp.float32),
            pltpu.VMEM((_NQ, _NB, _D), jnp.bfloat16),
            pltpu.VMEM((_NQ, _NB, _D), jnp.bfloat16),
            pltpu.VMEM((_NQ, _NB, _D), jnp.bfloat16),
            pltpu.VMEM((_NQ, _NB, _D), jnp.float32),
        ],
    )(x_tok, codebooks)
    quantized_out = qout.reshape(B, T, D).transpose(0, 2, 1)
    all_indices = idx6.T.reshape(B, T, _NQ)
    return quantized_out, all_indices, loss[0, 0], perp[0, 0]


# speedup vs baseline: 1.1285x; 1.1285x over previous
import jax
import jax.numpy as jnp
from jax.experimental import pallas as pl
from jax.experimental.pallas import tpu as pltpu

_NQ = 6
_NB = 1024
_D = 512
_TT = 512
_NTOK = 8192
_NT = _NTOK // _TT


def _rvq_body(x_ref, cb_ref, qout_ref, idx_ref, loss_ref, perp_ref,
              cnorm_ref, counts_ref, losssum_ref,
              chi_ref, cmid_ref, clo_ref):
    i = pl.program_id(0)

    @pl.when(i == 0)
    def _init():
        for q in range(_NQ):
            cq = cb_ref[q]
            cnorm_ref[q, :] = jnp.sum(cq * cq, axis=1)
            hi = cq.astype(jnp.bfloat16)
            rem = cq - hi.astype(jnp.float32)
            mid = rem.astype(jnp.bfloat16)
            lo = rem - mid.astype(jnp.float32)
            chi_ref[q] = hi
            cmid_ref[q] = mid
            clo_ref[q] = lo.astype(jnp.bfloat16)
        counts_ref[...] = jnp.zeros_like(counts_ref)
        losssum_ref[...] = jnp.zeros_like(losssum_ref)

    x = x_ref[...]
    res = x
    lanes = jax.lax.broadcasted_iota(jnp.int32, (_TT, _NB), 1)
    ones8 = jnp.ones((8, _TT), jnp.bfloat16)
    for q in range(_NQ):
        cq = cb_ref[q]
        xnorm = jnp.sum(res * res, axis=1, keepdims=True)
        if q > 0:
            losssum_ref[q - 1, :] += jnp.sum(xnorm)
        prod = jax.lax.dot_general(
            res, cq, (((1,), (1,)), ((), ())),
            preferred_element_type=jnp.float32)
        scores = (xnorm - 2.0 * prod) + cnorm_ref[q, :][None, :]
        idx = jnp.argmin(scores, axis=1).astype(jnp.int32)
        oh = (lanes == idx[:, None]).astype(jnp.bfloat16)
        dn = (((1,), (0,)), ((), ()))
        quant = (jax.lax.dot_general(oh, chi_ref[q], dn,
                                     preferred_element_type=jnp.float32)
                 + jax.lax.dot_general(oh, cmid_ref[q], dn,
                                       preferred_element_type=jnp.float32)
                 + jax.lax.dot_general(oh, clo_ref[q], dn,
                                       preferred_element_type=jnp.float32))
        res = res - quant
        idx_ref[q, :] = idx
        cnt = jax.lax.dot_general(ones8, oh, (((1,), (0,)), ((), ())),
                                  preferred_element_type=jnp.float32)
        counts_ref[q, :] += cnt[0, :]

    losssum_ref[_NQ - 1, :] += jnp.sum(res * res)
    qout_ref[...] = x - res

    @pl.when(i == _NT - 1)
    def _fin():
        rowmask = (jax.lax.broadcasted_iota(jnp.int32, (8, 1), 0) < _NQ)
        rowmask = rowmask.astype(jnp.float32)
        probs = counts_ref[...] * (1.0 / _NTOK)
        ent = -jnp.sum(probs * jnp.log(probs + 1e-10), axis=1, keepdims=True)
        perp = jnp.exp(ent) * rowmask
        perp_ref[...] = jnp.full((1, 1), jnp.sum(perp) / _NQ, jnp.float32)
        lsum = losssum_ref[...][:, 0:1] * rowmask
        mean_loss = jnp.sum(lsum) * (1.0 / (_NTOK * _D)) / _NQ
        loss_ref[...] = jnp.full((1, 1), mean_loss, jnp.float32)


def kernel(x, codebooks):
    B, D, T = x.shape
    x_tok = x.transpose(0, 2, 1).reshape(B * T, D)
    qout, idx6, loss, perp = pl.pallas_call(
        _rvq_body,
        grid=(_NT,),
        in_specs=[
            pl.BlockSpec((_TT, _D), lambda i: (i, 0)),
            pl.BlockSpec((_NQ, _NB, _D), lambda i: (0, 0, 0)),
        ],
        out_specs=[
            pl.BlockSpec((_TT, _D), lambda i: (i, 0)),
            pl.BlockSpec((_NQ, _TT), lambda i: (0, i)),
            pl.BlockSpec((1, 1), lambda i: (0, 0)),
            pl.BlockSpec((1, 1), lambda i: (0, 0)),
        ],
        out_shape=[
            jax.ShapeDtypeStruct((_NTOK, _D), jnp.float32),
            jax.ShapeDtypeStruct((_NQ, _NTOK), jnp.int32),
            jax.ShapeDtypeStruct((1, 1), jnp.float32),
            jax.ShapeDtypeStruct((1, 1), jnp.float32),
        ],
        scratch_shapes=[
            pltpu.VMEM((8, _NB), jnp.float32),
            pltpu.VMEM((8, _NB), jnp.float32),
            pltpu.VMEM((8, 128), jnp.float32),
            pltpu.VMEM((_NQ, _NB, _D), jnp.bfloat16),
            pltpu.VMEM((_NQ, _NB, _D), jnp.bfloat16),
            pltpu.VMEM((_NQ, _NB, _D), jnp.bfloat16),
        ],
    )(x_tok, codebooks)
    quantized_out = qout.reshape(B, T, D).transpose(0, 2, 1)
    all_indices = idx6.T.reshape(B, T, _NQ)
    return quantized_out, all_indices, loss[0, 0], perp[0, 0]
